# baseline (device time: 1081850 ns/iter reference)
import jax
import jax.numpy as jnp
from jax import lax
from jax.experimental import pallas as pl
from jax.experimental.pallas import tpu as pltpu

T = 4096
D = 2048
F = 4096
E_LOCAL = 4
N_TOK = 2 * T
C = 1152
F_TILE = 512
KT_TILE = 1024
A_ROWS = 32
VMEM_LIMIT = 60 * 1024 * 1024


def _neighbor():
    my_x = lax.axis_index("x")
    my_y = lax.axis_index("y")
    my_z = lax.axis_index("z")
    return my_y, (my_x, 1 - my_y, my_z)


def _exchange_body(x_ref, a_ref, xall_ref, aall_ref,
                   sx_send, sx_recv, sa_send, sa_recv):
    my_y, nbr = _neighbor()

    barrier = pltpu.get_barrier_semaphore()
    pl.semaphore_signal(barrier, inc=1, device_id=nbr,
                        device_id_type=pl.DeviceIdType.MESH)
    pl.semaphore_wait(barrier, 1)

    row0 = my_y * T
    arow0 = my_y * A_ROWS
    xall_ref[pl.ds(row0, T), :] = x_ref[...]
    aall_ref[pl.ds(arow0, A_ROWS), :] = a_ref[...]

    rx = pltpu.make_async_remote_copy(
        src_ref=xall_ref.at[pl.ds(row0, T)],
        dst_ref=xall_ref.at[pl.ds(row0, T)],
        send_sem=sx_send, recv_sem=sx_recv,
        device_id=nbr, device_id_type=pl.DeviceIdType.MESH,
    )
    ra = pltpu.make_async_remote_copy(
        src_ref=aall_ref.at[pl.ds(arow0, A_ROWS)],
        dst_ref=aall_ref.at[pl.ds(arow0, A_ROWS)],
        send_sem=sa_send, recv_sem=sa_recv,
        device_id=nbr, device_id_type=pl.DeviceIdType.MESH,
    )
    rx.start()
    ra.start()
    rx.wait()
    ra.wait()


def _exchange(x_bf16, assign2d):
    return pl.pallas_call(
        _exchange_body,
        out_shape=(
            jax.ShapeDtypeStruct((N_TOK, D), jnp.bfloat16),
            jax.ShapeDtypeStruct((2 * A_ROWS, 128), jnp.int32),
        ),
        in_specs=[
            pl.BlockSpec(memory_space=pltpu.VMEM),
            pl.BlockSpec(memory_space=pltpu.VMEM),
        ],
        out_specs=(
            pl.BlockSpec(memory_space=pltpu.VMEM),
            pl.BlockSpec(memory_space=pltpu.VMEM),
        ),
        scratch_shapes=[
            pltpu.SemaphoreType.DMA,
            pltpu.SemaphoreType.DMA,
            pltpu.SemaphoreType.DMA,
            pltpu.SemaphoreType.DMA,
        ],
        compiler_params=pltpu.CompilerParams(
            collective_id=0, vmem_limit_bytes=VMEM_LIMIT
        ),
    )(x_bf16, assign2d)


def _gather_body(seg_ref, rank_ref, x_ref, xg_ref, acc_ref):
    e = pl.program_id(0)
    kt = pl.program_id(1)
    n_kt = pl.num_programs(1)
    start_e = seg_ref[e]

    rank_blk = rank_ref[0, :]
    slot = jax.lax.broadcasted_iota(jnp.int32, (C, KT_TILE), 0) + start_e
    p = (slot == rank_blk[None, :]).astype(jnp.bfloat16)
    part = jnp.dot(p, x_ref[...], preferred_element_type=jnp.float32)

    @pl.when(kt == 0)
    def _():
        acc_ref[...] = part

    @pl.when(kt > 0)
    def _():
        acc_ref[...] += part

    @pl.when(kt == n_kt - 1)
    def _():
        xg_ref[0] = acc_ref[...].astype(jnp.bfloat16)


def _gather(seg, rank2d, x_all):
    n_kt = N_TOK // KT_TILE
    return pl.pallas_call(
        _gather_body,
        grid=(E_LOCAL, n_kt),
        out_shape=jax.ShapeDtypeStruct((E_LOCAL, C, D), jnp.bfloat16),
        in_specs=[
            pl.BlockSpec(memory_space=pltpu.SMEM),
            pl.BlockSpec((1, KT_TILE), lambda e, kt: (0, kt)),
            pl.BlockSpec((KT_TILE, D), lambda e, kt: (kt, 0)),
        ],
        out_specs=pl.BlockSpec((1, C, D), lambda e, kt: (e, 0, 0)),
        scratch_shapes=[pltpu.VMEM((C, D), jnp.float32)],
        compiler_params=pltpu.CompilerParams(vmem_limit_bytes=VMEM_LIMIT),
    )(seg, rank2d, x_all)


def _moe_body(xg_ref, w1_ref, w2_ref, og_ref, acc_ref):
    fb = pl.program_id(1)
    n_fb = pl.num_programs(1)
    xg = xg_ref[0]
    w1 = w1_ref[0].astype(jnp.bfloat16)
    h = jnp.maximum(
        jnp.dot(xg, w1, preferred_element_type=jnp.float32), 0.0
    ).astype(jnp.bfloat16)
    w2 = w2_ref[0].astype(jnp.bfloat16)
    p = jnp.dot(h, w2, preferred_element_type=jnp.float32)

    @pl.when(fb == 0)
    def _():
        acc_ref[...] = p

    @pl.when(fb > 0)
    def _():
        acc_ref[...] += p

    @pl.when(fb == n_fb - 1)
    def _():
        og_ref[0] = acc_ref[...].astype(jnp.bfloat16)


def _moe(xg, w1, w2):
    n_fb = F // F_TILE
    return pl.pallas_call(
        _moe_body,
        grid=(E_LOCAL, n_fb),
        out_shape=jax.ShapeDtypeStruct((E_LOCAL, C, D), jnp.bfloat16),
        in_specs=[
            pl.BlockSpec((1, C, D), lambda e, fb: (e, 0, 0)),
            pl.BlockSpec((1, D, F_TILE), lambda e, fb: (e, 0, fb)),
            pl.BlockSpec((1, F_TILE, D), lambda e, fb: (e, fb, 0)),
        ],
        out_specs=pl.BlockSpec((1, C, D), lambda e, fb: (e, 0, 0)),
        scratch_shapes=[pltpu.VMEM((C, D), jnp.float32)],
        compiler_params=pltpu.CompilerParams(vmem_limit_bytes=VMEM_LIMIT),
    )(xg, w1, w2)


def _scatter_body(seg_ref, rank_ref, og_ref, out_ref, acc_ref):
    tb = pl.program_id(0)
    e = pl.program_id(1)
    start_e = seg_ref[e]
    end_e = seg_ref[e + 1]

    rank_col = rank_ref[0, :][:, None]
    slot = jax.lax.broadcasted_iota(jnp.int32, (KT_TILE, C), 1) + start_e
    pt = jnp.where(
        (slot == rank_col) & (rank_col < end_e), 1.0, 0.0
    ).astype(jnp.bfloat16)
    part = jnp.dot(pt, og_ref[0], preferred_element_type=jnp.float32)

    @pl.when(e == 0)
    def _():
        acc_ref[...] = part

    @pl.when(e > 0)
    def _():
        acc_ref[...] += part

    @pl.when(e == E_LOCAL - 1)
    def _():
        out_ref[...] = acc_ref[...].astype(jnp.bfloat16)


def _scatter(seg, rank2d, og):
    n_tb = N_TOK // KT_TILE
    return pl.pallas_call(
        _scatter_body,
        grid=(n_tb, E_LOCAL),
        out_shape=jax.ShapeDtypeStruct((N_TOK, D), jnp.bfloat16),
        in_specs=[
            pl.BlockSpec(memory_space=pltpu.SMEM),
            pl.BlockSpec((1, KT_TILE), lambda tb, e: (0, tb)),
            pl.BlockSpec((1, C, D), lambda tb, e: (e, 0, 0)),
        ],
        out_specs=pl.BlockSpec((KT_TILE, D), lambda tb, e: (tb, 0)),
        scratch_shapes=[pltpu.VMEM((KT_TILE, D), jnp.float32)],
        compiler_params=pltpu.CompilerParams(vmem_limit_bytes=VMEM_LIMIT),
    )(seg, rank2d, og)


def _combine_body(in_ref, out_ref, s_send, s_recv):
    my_y, nbr = _neighbor()

    barrier = pltpu.get_barrier_semaphore()
    pl.semaphore_signal(barrier, inc=1, device_id=nbr,
                        device_id_type=pl.DeviceIdType.MESH)
    pl.semaphore_wait(barrier, 1)

    other0 = (1 - my_y) * T
    r = pltpu.make_async_remote_copy(
        src_ref=in_ref.at[pl.ds(other0, T)],
        dst_ref=out_ref,
        send_sem=s_send, recv_sem=s_recv,
        device_id=nbr, device_id_type=pl.DeviceIdType.MESH,
    )
    r.start()
    r.wait()

    mine0 = my_y * T
    out_ref[...] = out_ref[...] + in_ref[pl.ds(mine0, T), :]


def _combine(out_all):
    return pl.pallas_call(
        _combine_body,
        out_shape=jax.ShapeDtypeStruct((T, D), jnp.bfloat16),
        in_specs=[pl.BlockSpec(memory_space=pltpu.VMEM)],
        out_specs=pl.BlockSpec(memory_space=pltpu.VMEM),
        scratch_shapes=[
            pltpu.SemaphoreType.DMA,
            pltpu.SemaphoreType.DMA,
        ],
        compiler_params=pltpu.CompilerParams(
            collective_id=1, vmem_limit_bytes=VMEM_LIMIT
        ),
    )(out_all)


def kernel(x, assign, W1, W2):
    x_all, a2d = _exchange(x.astype(jnp.bfloat16), assign.reshape(A_ROWS, 128))
    assign_all = a2d.reshape(N_TOK)

    my_y = lax.axis_index("y")
    sort_idx = jnp.argsort(assign_all)
    rank = jnp.argsort(sort_idx)
    sorted_a = jnp.sort(assign_all)
    starts = jnp.searchsorted(sorted_a, jnp.arange(9, dtype=assign_all.dtype))
    seg = lax.dynamic_slice(starts, (my_y * E_LOCAL,), (E_LOCAL + 1,))
    seg = seg.astype(jnp.int32)
    rank2d = rank.astype(jnp.int32).reshape(1, N_TOK)

    xg = _gather(seg, rank2d, x_all)
    og = _moe(xg, W1, W2)
    out_all = _scatter(seg, rank2d, og)

    out = _combine(out_all)
    return out.astype(jnp.float32)


# device time: 697011 ns/iter; 1.5521x vs baseline; 1.5521x over previous
import jax
import jax.numpy as jnp
from jax import lax
from jax.experimental import pallas as pl
from jax.experimental.pallas import tpu as pltpu

T = 4096
D = 2048
F = 4096
E_LOCAL = 4
N_TOK = 2 * T
C = 1152
F_TILE = 512
KT_TILE = 1024
N_BLK = N_TOK // KT_TILE
N_HALF = N_BLK // 2
A_ROWS = 32
VMEM_LIMIT = 62 * 1024 * 1024


def _neighbor():
    my_x = lax.axis_index("x")
    my_y = lax.axis_index("y")
    my_z = lax.axis_index("z")
    return my_y, (my_x, 1 - my_y, my_z)


def _aexch_body(a_ref, aall_ref, s_send, s_recv):
    my_y, nbr = _neighbor()
    barrier = pltpu.get_barrier_semaphore()
    pl.semaphore_signal(barrier, inc=1, device_id=nbr,
                        device_id_type=pl.DeviceIdType.MESH)
    pl.semaphore_wait(barrier, 1)

    row0 = my_y * A_ROWS
    aall_ref[pl.ds(row0, A_ROWS), :] = a_ref[...]
    r = pltpu.make_async_remote_copy(
        src_ref=aall_ref.at[pl.ds(row0, A_ROWS)],
        dst_ref=aall_ref.at[pl.ds(row0, A_ROWS)],
        send_sem=s_send, recv_sem=s_recv,
        device_id=nbr, device_id_type=pl.DeviceIdType.MESH,
    )
    r.start()
    r.wait()


def _assign_exchange(assign2d):
    return pl.pallas_call(
        _aexch_body,
        out_shape=jax.ShapeDtypeStruct((2 * A_ROWS, 128), jnp.int32),
        in_specs=[pl.BlockSpec(memory_space=pltpu.VMEM)],
        out_specs=pl.BlockSpec(memory_space=pltpu.VMEM),
        scratch_shapes=[pltpu.SemaphoreType.DMA, pltpu.SemaphoreType.DMA],
        compiler_params=pltpu.CompilerParams(collective_id=0),
    )(assign2d)


def _gather_body(seg_ref, ord_ref, rank_ref, xany_ref, xblk_ref, xg_ref,
                 acc_ref, xrem_ref, sems_send, sems_recv):
    i = pl.program_id(0)
    e = pl.program_id(1)
    _, nbr = _neighbor()

    rdma = [
        pltpu.make_async_remote_copy(
            src_ref=xany_ref.at[pl.ds(k * KT_TILE, KT_TILE)],
            dst_ref=xrem_ref.at[k],
            send_sem=sems_send.at[k], recv_sem=sems_recv.at[k],
            device_id=nbr, device_id_type=pl.DeviceIdType.MESH,
        )
        for k in range(N_HALF)
    ]

    @pl.when((i == 0) & (e == 0))
    def _():
        barrier = pltpu.get_barrier_semaphore()
        pl.semaphore_signal(barrier, inc=1, device_id=nbr,
                            device_id_type=pl.DeviceIdType.MESH)
        pl.semaphore_wait(barrier, 1)
        for k in range(N_HALF):
            rdma[k].start()

    for k in range(N_HALF):
        @pl.when((i == N_HALF + k) & (e == 0))
        def _(k=k):
            rdma[k].wait_recv()

    start_e = seg_ref[e]
    rank_blk = rank_ref[0, :]
    slot = jax.lax.broadcasted_iota(jnp.int32, (C, KT_TILE), 0) + start_e
    p = (slot == rank_blk[None, :]).astype(jnp.bfloat16)

    def _accum(xblk):
        part = jnp.dot(p, xblk, preferred_element_type=jnp.float32)
        pb = part.astype(jnp.bfloat16)

        @pl.when(i == 0)
        def _():
            acc_ref[e] = pb

        @pl.when(i > 0)
        def _():
            acc_ref[e] += pb

    @pl.when(i < N_HALF)
    def _():
        _accum(xblk_ref[...])

    @pl.when(i >= N_HALF)
    def _():
        _accum(xrem_ref[i - N_HALF])

    @pl.when(i == N_BLK - 1)
    def _():
        xg_ref[0] = acc_ref[e]

    @pl.when((i == N_BLK - 1) & (e == E_LOCAL - 1))
    def _():
        for k in range(N_HALF):
            rdma[k].wait_send()


def _gather(seg, ord_, rank2d, x_bf16):
    grid_spec = pltpu.PrefetchScalarGridSpec(
        num_scalar_prefetch=2,
        grid=(N_BLK, E_LOCAL),
        in_specs=[
            pl.BlockSpec((1, KT_TILE), lambda i, e, seg, ordr: (0, ordr[i])),
            pl.BlockSpec(memory_space=pl.ANY),
            pl.BlockSpec(
                (KT_TILE, D),
                lambda i, e, seg, ordr: (jnp.where(i < N_HALF, i, N_HALF - 1), 0),
            ),
        ],
        out_specs=pl.BlockSpec((1, C, D), lambda i, e, seg, ordr: (e, 0, 0)),
        scratch_shapes=[
            pltpu.VMEM((E_LOCAL, C, D), jnp.bfloat16),
            pltpu.VMEM((N_HALF, KT_TILE, D), jnp.bfloat16),
            pltpu.SemaphoreType.DMA((N_HALF,)),
            pltpu.SemaphoreType.DMA((N_HALF,)),
        ],
    )
    return pl.pallas_call(
        _gather_body,
        grid_spec=grid_spec,
        out_shape=jax.ShapeDtypeStruct((E_LOCAL, C, D), jnp.bfloat16),
        compiler_params=pltpu.CompilerParams(
            collective_id=1, vmem_limit_bytes=VMEM_LIMIT
        ),
    )(seg, ord_, rank2d, x_bf16, x_bf16)


def _moe_body(xg_ref, w1_ref, w2_ref, og_ref, acc_ref):
    fb = pl.program_id(1)
    n_fb = pl.num_programs(1)
    xg = xg_ref[0]
    w1 = w1_ref[0].astype(jnp.bfloat16)
    h = jnp.maximum(
        jnp.dot(xg, w1, preferred_element_type=jnp.float32), 0.0
    ).astype(jnp.bfloat16)
    w2 = w2_ref[0].astype(jnp.bfloat16)
    p = jnp.dot(h, w2, preferred_element_type=jnp.float32)

    @pl.when(fb == 0)
    def _():
        acc_ref[...] = p

    @pl.when(fb > 0)
    def _():
        acc_ref[...] += p

    @pl.when(fb == n_fb - 1)
    def _():
        og_ref[0] = acc_ref[...].astype(jnp.bfloat16)


def _moe(xg, w1, w2):
    n_fb = F // F_TILE
    return pl.pallas_call(
        _moe_body,
        grid=(E_LOCAL, n_fb),
        out_shape=jax.ShapeDtypeStruct((E_LOCAL, C, D), jnp.bfloat16),
        in_specs=[
            pl.BlockSpec((1, C, D), lambda e, fb: (e, 0, 0)),
            pl.BlockSpec((1, D, F_TILE), lambda e, fb: (e, 0, fb)),
            pl.BlockSpec((1, F_TILE, D), lambda e, fb: (e, fb, 0)),
        ],
        out_specs=pl.BlockSpec((1, C, D), lambda e, fb: (e, 0, 0)),
        scratch_shapes=[pltpu.VMEM((C, D), jnp.float32)],
        compiler_params=pltpu.CompilerParams(vmem_limit_bytes=VMEM_LIMIT),
    )(xg, w1, w2)


def _scatter_body(seg_ref, ord_ref, rank_ref, og_ref, out_ref,
                  acc_ref, stage_ref, recv_ref, sems_send, sems_recv):
    tb = pl.program_id(0)
    e = pl.program_id(1)
    _, nbr = _neighbor()

    rdma = [
        pltpu.make_async_remote_copy(
            src_ref=stage_ref.at[k],
            dst_ref=recv_ref.at[k],
            send_sem=sems_send.at[k], recv_sem=sems_recv.at[k],
            device_id=nbr, device_id_type=pl.DeviceIdType.MESH,
        )
        for k in range(N_HALF)
    ]

    @pl.when((tb == 0) & (e == 0))
    def _():
        barrier = pltpu.get_barrier_semaphore()
        pl.semaphore_signal(barrier, inc=1, device_id=nbr,
                            device_id_type=pl.DeviceIdType.MESH)
        pl.semaphore_wait(barrier, 1)

    start_e = seg_ref[e]
    end_e = seg_ref[e + 1]
    rank_col = rank_ref[0, :][:, None]
    slot = jax.lax.broadcasted_iota(jnp.int32, (KT_TILE, C), 1) + start_e
    pt = ((slot == rank_col) & (rank_col < end_e)).astype(jnp.bfloat16)
    part = jnp.dot(pt, og_ref[0], preferred_element_type=jnp.float32)
    pb = part.astype(jnp.bfloat16)

    @pl.when(e == 0)
    def _():
        acc_ref[...] = pb

    @pl.when(e > 0)
    def _():
        acc_ref[...] += pb

    @pl.when(e == E_LOCAL - 1)
    def _():
        for k in range(N_HALF):
            @pl.when(tb == k)
            def _(k=k):
                stage_ref[k] = acc_ref[...]
                rdma[k].start()

        for k in range(N_HALF):
            @pl.when(tb == N_HALF + k)
            def _(k=k):
                rdma[k].wait_recv()
                out_ref[...] = acc_ref[...] + recv_ref[k]

    @pl.when((tb == N_BLK - 1) & (e == E_LOCAL - 1))
    def _():
        for k in range(N_HALF):
            rdma[k].wait_send()


def _scatter(seg, ord_, rank2d, og):
    grid_spec = pltpu.PrefetchScalarGridSpec(
        num_scalar_prefetch=2,
        grid=(N_BLK, E_LOCAL),
        in_specs=[
            pl.BlockSpec((1, KT_TILE), lambda tb, e, seg, ordr: (0, ordr[tb])),
            pl.BlockSpec((1, C, D), lambda tb, e, seg, ordr: (e, 0, 0)),
        ],
        out_specs=pl.BlockSpec(
            (KT_TILE, D),
            lambda tb, e, seg, ordr: (jnp.where(tb < N_HALF, 0, tb - N_HALF), 0),
        ),
        scratch_shapes=[
            pltpu.VMEM((KT_TILE, D), jnp.bfloat16),
            pltpu.VMEM((N_HALF, KT_TILE, D), jnp.bfloat16),
            pltpu.VMEM((N_HALF, KT_TILE, D), jnp.bfloat16),
            pltpu.SemaphoreType.DMA((N_HALF,)),
            pltpu.SemaphoreType.DMA((N_HALF,)),
        ],
    )
    return pl.pallas_call(
        _scatter_body,
        grid_spec=grid_spec,
        out_shape=jax.ShapeDtypeStruct((T, D), jnp.bfloat16),
        compiler_params=pltpu.CompilerParams(
            collective_id=2, vmem_limit_bytes=VMEM_LIMIT
        ),
    )(seg, ord_, rank2d, og)


def kernel(x, assign, W1, W2):
    my_y = lax.axis_index("y")

    a2d = _assign_exchange(assign.reshape(A_ROWS, 128))
    assign_all = a2d.reshape(N_TOK)

    sort_idx = jnp.argsort(assign_all)
    rank = jnp.argsort(sort_idx)
    sorted_a = jnp.sort(assign_all)
    starts = jnp.searchsorted(sorted_a, jnp.arange(9, dtype=assign_all.dtype))
    seg = lax.dynamic_slice(starts, (my_y * E_LOCAL,), (E_LOCAL + 1,))
    seg = seg.astype(jnp.int32)
    rank2d = rank.astype(jnp.int32).reshape(1, N_TOK)

    blocks_mine = my_y * N_HALF + jnp.arange(N_HALF, dtype=jnp.int32)
    blocks_nbr = (1 - my_y) * N_HALF + jnp.arange(N_HALF, dtype=jnp.int32)
    ord_g = jnp.concatenate([blocks_mine, blocks_nbr])
    ord_s = jnp.concatenate([blocks_nbr, blocks_mine])

    xg = _gather(seg, ord_g, rank2d, x.astype(jnp.bfloat16))
    og = _moe(xg, W1, W2)
    out = _scatter(seg, ord_s, rank2d, og)
    return out.astype(jnp.float32)
